# Initial kernel scaffold; baseline (speedup 1.0000x reference)
import jax
import jax.numpy as jnp
from jax.experimental import pallas as pl


def kernel(coords, features, mlp1_w, mlp1_b, lse1_w, lse1_b, lse1_g, lse1_bt, pool1_sw, pool1_sb, pool1_ow, pool1_ob, pool1_g, pool1_bt, lse2_w, lse2_b, lse2_g, lse2_bt, pool2_sw, pool2_sb, pool2_ow, pool2_ob, pool2_g, pool2_bt, mlp2_w, mlp2_b, short_w, short_b, short_g, short_bt):
    raise NotImplementedError("write your pallas kernel here")


# speedup vs baseline: 4.3865x; 4.3865x over previous
import functools

import jax
import jax.numpy as jnp
from jax.experimental import pallas as pl
from jax.experimental.pallas import tpu as pltpu

F32 = jnp.float32
HIGH = jax.lax.Precision.HIGHEST
BIG = 3.0e38


def _dot(a, b):
    return jax.lax.dot_general(a, b, (((1,), (0,)), ((), ())),
                               preferred_element_type=F32, precision=HIGH)


def _dott(a, b):
    return jax.lax.dot_general(a, b, (((0,), (0,)), ((), ())),
                               preferred_element_type=F32, precision=HIGH)


def _p1_body(K, Q, N, keys_ref, kt_ref, concat_ref, mom_ref, nb_scr, d2_scr):
    b = pl.program_id(0)
    qi = pl.program_id(1)
    keys = keys_ref[0]
    kt = kt_ref[0]
    q = keys_ref[0, pl.ds(qi * Q, Q), :]
    sqq = jnp.sum(q * q, axis=1, keepdims=True)
    sqk = jnp.sum(kt * kt, axis=0, keepdims=True)
    gram = jax.lax.dot_general(q, kt, (((1,), (0,)), ((), ())),
                               preferred_element_type=F32,
                               precision=jax.lax.Precision.DEFAULT)
    d2 = sqq + sqk - 2.0 * gram
    d2_scr[...] = jnp.maximum(d2, 0.0)

    def body(k, carry):
        d2v = d2_scr[...]
        iota = jax.lax.broadcasted_iota(jnp.int32, (Q, N), 1)
        m = jnp.min(d2v, axis=1, keepdims=True)
        sel = jnp.min(jnp.where(d2v == m, iota, N), axis=1,
                      keepdims=True)
        oh = iota == sel
        ohf = jnp.where(oh, 1.0, 0.0).astype(F32)
        nb = _dot(ohf, keys)
        row = jnp.concatenate(
            [nb[:, 0:3], jnp.sqrt(m + 1e-12), jnp.zeros((Q, 4), F32)],
            axis=1)
        nb_scr[pl.ds(k, 1)] = row[None]
        d2_scr[...] = jnp.where(oh, BIG, d2v)
        return carry

    jax.lax.fori_loop(0, K, body, 0)
    m2 = jnp.zeros((16, 16), F32)
    sc = jnp.zeros((1, 16), F32)
    for k in range(K):
        rk = nb_scr[k]
        catk = jnp.concatenate(
            [q[:, 0:3], rk[:, 0:3], q[:, 0:3] - rk[:, 0:3], rk[:, 3:4],
             jnp.zeros((Q, 6), F32)], axis=1)
        concat_ref[0, :, k, :] = catk
        m2 = m2 + _dott(catk, catk)
        sc = sc + jnp.sum(catk, axis=0, keepdims=True)

    @pl.when(jnp.logical_and(b == 0, qi == 0))
    def _():
        mom_ref[...] = jnp.zeros_like(mom_ref)

    mom_ref[0:16, :] += m2
    mom_ref[16:17, :] += sc


def _p2_body(K, Q, concat_ref, feat_ref, w1_ref, b1_ref, w2_ref, b2_ref,
             sw1_ref, sw2_ref, m1w_ref, m1b_ref, o1g_ref, o1f_ref, o1b_ref,
             shw_ref, shb_ref,
             g2_ref, y1_ref, ys_ref, st1_ref, sts_ref):
    b = pl.program_id(0)
    qi = pl.program_id(1)
    flat = jnp.reshape(concat_ref[0], (Q * K, 16))
    enc1 = jnp.maximum(_dot(flat, w1_ref[...]) + b1_ref[...], 0.0)
    enc2 = jnp.maximum(_dot(flat, w2_ref[...]) + b2_ref[...], 0.0)

    def pool(enc, sw):
        a = _dot(enc, sw)
        ar = jnp.reshape(a, (Q, K, 64))
        er = jnp.reshape(enc, (Q, K, 64))
        mx = jnp.max(ar, axis=1, keepdims=True)
        ex = jnp.exp(ar - mx)
        s = ex / jnp.sum(ex, axis=1, keepdims=True)
        return jnp.sum(s * er, axis=1)

    g1 = pool(enc1, sw1_ref[...])
    g2 = pool(enc2, sw2_ref[...])
    g2_ref[0] = g2
    f = feat_ref[0]
    x0 = _dot(f, m1w_ref[...]) + m1b_ref[...]
    x0 = jnp.where(x0 >= 0, x0, 0.2 * x0)
    y1 = _dot(g1, o1g_ref[...]) + _dot(x0, o1f_ref[...]) + o1b_ref[...]
    y1_ref[0] = y1
    ys = _dot(f, shw_ref[...]) + shb_ref[...]
    ys_ref[0] = ys

    @pl.when(jnp.logical_and(b == 0, qi == 0))
    def _():
        st1_ref[...] = jnp.zeros_like(st1_ref)
        sts_ref[...] = jnp.zeros_like(sts_ref)

    st1_ref[0:1, :] += jnp.sum(y1, axis=0, keepdims=True)
    st1_ref[1:2, :] += jnp.sum(y1 * y1, axis=0, keepdims=True)
    sts_ref[0:1, :] += jnp.sum(ys, axis=0, keepdims=True)
    sts_ref[1:2, :] += jnp.sum(ys * ys, axis=0, keepdims=True)


def _p3_body(y1_ref, g2_ref, o2g_ref, o2f_ref, o2b_ref, sc1_ref, sh1_ref,
             y2_ref, st2_ref):
    b = pl.program_id(0)
    qi = pl.program_id(1)
    x1 = jnp.maximum(y1_ref[0] * sc1_ref[...] + sh1_ref[...], 0.0)
    y2 = _dot(g2_ref[0], o2g_ref[...]) + _dot(x1, o2f_ref[...]) + o2b_ref[...]
    y2_ref[0] = y2

    @pl.when(jnp.logical_and(b == 0, qi == 0))
    def _():
        st2_ref[...] = jnp.zeros_like(st2_ref)

    st2_ref[0:1, :] += jnp.sum(y2, axis=0, keepdims=True)
    st2_ref[1:2, :] += jnp.sum(y2 * y2, axis=0, keepdims=True)


def _p4_body(y2_ref, ys_ref, m2w_ref, m2b_ref, sc2_ref, sh2_ref, scs_ref,
             shs_ref, out_ref):
    x2 = jnp.maximum(y2_ref[0] * sc2_ref[...] + sh2_ref[...], 0.0)
    o = _dot(x2, m2w_ref[...]) + m2b_ref[...] + ys_ref[0] * scs_ref[...] \
        + shs_ref[...]
    out_ref[0] = jnp.where(o >= 0, o, 0.01 * o)


def kernel(coords, features, mlp1_w, mlp1_b, lse1_w, lse1_b, lse1_g, lse1_bt,
           pool1_sw, pool1_sb, pool1_ow, pool1_ob, pool1_g, pool1_bt,
           lse2_w, lse2_b, lse2_g, lse2_bt, pool2_sw, pool2_sb, pool2_ow,
           pool2_ob, pool2_g, pool2_bt, mlp2_w, mlp2_b, short_w, short_b,
           short_g, short_bt):
    B, N, _ = coords.shape
    K = 16
    h = mlp1_w.shape[0]
    d_in = mlp1_w.shape[1]
    d_out = pool1_sw.shape[0]
    d_fin = mlp2_w.shape[0]
    eps = 1e-6

    coords_pad = jnp.concatenate(
        [coords, jnp.zeros((B, N, 8 - coords.shape[2]), F32)], axis=2)
    coords_t = jnp.transpose(coords_pad, (0, 2, 1))
    feat_t = jnp.transpose(features[:, :, :, 0], (0, 2, 1))

    Q1 = 512
    nb1 = N // Q1
    p1 = pl.pallas_call(
        functools.partial(_p1_body, K, Q1, N),
        grid=(B, nb1),
        in_specs=[
            pl.BlockSpec((1, N, 8), lambda b, q: (b, 0, 0)),
            pl.BlockSpec((1, 8, N), lambda b, q: (b, 0, 0)),
        ],
        out_specs=[
            pl.BlockSpec((1, Q1, K, 16), lambda b, q: (b, q, 0, 0)),
            pl.BlockSpec((24, 16), lambda b, q: (0, 0)),
        ],
        out_shape=[
            jax.ShapeDtypeStruct((B, N, K, 16), F32),
            jax.ShapeDtypeStruct((24, 16), F32),
        ],
        scratch_shapes=[
            pltpu.VMEM((K, Q1, 8), F32),
            pltpu.VMEM((Q1, N), F32),
        ],
    )
    concat, mom = p1(coords_pad, coords_t)

    M = B * N * K
    mu = mom[16, 0:10] / M
    m2 = mom[0:10, 0:10] / M
    cc = m2 - jnp.outer(mu, mu)

    def fold(w, bb, g, bt):
        mean = w @ mu + bb
        var = jnp.sum((w @ cc) * w, axis=1)
        sc = g / jnp.sqrt(var + eps)
        weff = jnp.zeros((16, h), F32).at[0:10, :].set((w * sc[:, None]).T)
        beff = (bb - mean) * sc + bt
        return weff, beff[None, :]

    w1eff, b1eff = fold(lse1_w, lse1_b, lse1_g, lse1_bt)
    w2eff, b2eff = fold(lse2_w, lse2_b, lse2_g, lse2_bt)

    Q2 = 512
    nb2 = N // Q2
    cst = lambda b, q: (0, 0)
    p2 = pl.pallas_call(
        functools.partial(_p2_body, K, Q2),
        grid=(B, nb2),
        in_specs=[
            pl.BlockSpec((1, Q2, K, 16), lambda b, q: (b, q, 0, 0)),
            pl.BlockSpec((1, Q2, d_in), lambda b, q: (b, q, 0)),
            pl.BlockSpec((16, h), cst), pl.BlockSpec((1, h), cst),
            pl.BlockSpec((16, h), cst), pl.BlockSpec((1, h), cst),
            pl.BlockSpec((h, h), cst), pl.BlockSpec((h, h), cst),
            pl.BlockSpec((d_in, h), cst), pl.BlockSpec((1, h), cst),
            pl.BlockSpec((h, h), cst), pl.BlockSpec((h, h), cst),
            pl.BlockSpec((1, h), cst),
            pl.BlockSpec((d_in, d_fin), cst), pl.BlockSpec((1, d_fin), cst),
        ],
        out_specs=[
            pl.BlockSpec((1, Q2, h), lambda b, q: (b, q, 0)),
            pl.BlockSpec((1, Q2, h), lambda b, q: (b, q, 0)),
            pl.BlockSpec((1, Q2, d_fin), lambda b, q: (b, q, 0)),
            pl.BlockSpec((8, h), cst),
            pl.BlockSpec((8, d_fin), cst),
        ],
        out_shape=[
            jax.ShapeDtypeStruct((B, N, h), F32),
            jax.ShapeDtypeStruct((B, N, h), F32),
            jax.ShapeDtypeStruct((B, N, d_fin), F32),
            jax.ShapeDtypeStruct((8, h), F32),
            jax.ShapeDtypeStruct((8, d_fin), F32),
        ],
    )
    g2, y1, ys, st1, sts = p2(
        concat, feat_t,
        w1eff, b1eff, w2eff, b2eff,
        pool1_sw[0:h, 0:h].T, pool2_sw[0:h, 0:h].T,
        mlp1_w.T, mlp1_b[None, :],
        pool1_ow[:, 0:h].T, pool1_ow[:, h:].T, pool1_ob[None, :],
        short_w.T, short_b[None, :])

    def bnstats(st, g, bt, cnt):
        mean = st[0] / cnt
        var = st[1] / cnt - mean * mean
        sc = g / jnp.sqrt(var + eps)
        return sc[None, :], (bt - mean * sc)[None, :]

    sc1, sh1 = bnstats(st1, pool1_g, pool1_bt, B * N)
    scs, shs = bnstats(sts, short_g, short_bt, B * N)

    Q3 = 2048
    nb3 = N // Q3
    p3 = pl.pallas_call(
        _p3_body,
        grid=(B, nb3),
        in_specs=[
            pl.BlockSpec((1, Q3, h), lambda b, q: (b, q, 0)),
            pl.BlockSpec((1, Q3, h), lambda b, q: (b, q, 0)),
            pl.BlockSpec((h, d_out), cst), pl.BlockSpec((h, d_out), cst),
            pl.BlockSpec((1, d_out), cst),
            pl.BlockSpec((1, h), cst), pl.BlockSpec((1, h), cst),
        ],
        out_specs=[
            pl.BlockSpec((1, Q3, d_out), lambda b, q: (b, q, 0)),
            pl.BlockSpec((8, d_out), cst),
        ],
        out_shape=[
            jax.ShapeDtypeStruct((B, N, d_out), F32),
            jax.ShapeDtypeStruct((8, d_out), F32),
        ],
    )
    y2, st2 = p3(y1, g2, pool2_ow[:, 0:h].T, pool2_ow[:, h:].T,
                 pool2_ob[None, :], sc1, sh1)

    sc2, sh2 = bnstats(st2, pool2_g, pool2_bt, B * N)

    p4 = pl.pallas_call(
        _p4_body,
        grid=(B, nb3),
        in_specs=[
            pl.BlockSpec((1, Q3, d_out), lambda b, q: (b, q, 0)),
            pl.BlockSpec((1, Q3, d_fin), lambda b, q: (b, q, 0)),
            pl.BlockSpec((d_out, d_fin), cst), pl.BlockSpec((1, d_fin), cst),
            pl.BlockSpec((1, d_out), cst), pl.BlockSpec((1, d_out), cst),
            pl.BlockSpec((1, d_fin), cst), pl.BlockSpec((1, d_fin), cst),
        ],
        out_specs=[pl.BlockSpec((1, Q3, d_fin), lambda b, q: (b, q, 0))],
        out_shape=[jax.ShapeDtypeStruct((B, N, d_fin), F32)],
    )
    (out,) = p4(y2, ys, mlp2_w.T, mlp2_b[None, :], sc2, sh2, scs, shs)

    return jnp.transpose(out, (0, 2, 1))[:, :, :, None]


# speedup vs baseline: 7.4679x; 1.7025x over previous
import functools

import jax
import jax.numpy as jnp
from jax.experimental import pallas as pl
from jax.experimental.pallas import tpu as pltpu

F32 = jnp.float32
BF16 = jnp.bfloat16
BIG = 3.0e38


def _dot(a, b):
    return jax.lax.dot_general(a, b, (((1,), (0,)), ((), ())),
                               preferred_element_type=F32,
                               precision=jax.lax.Precision.DEFAULT)


def _dott(a, b):
    return jax.lax.dot_general(a, b, (((0,), (0,)), ((), ())),
                               preferred_element_type=F32,
                               precision=jax.lax.Precision.HIGHEST)


def _p1_body(K, Q, N, keys_ref, kt_ref, khi_ref, kmid_ref, klo_ref,
             concat_ref, mom_ref, d2_scr):
    b = pl.program_id(0)
    qi = pl.program_id(1)
    khi = khi_ref[0]
    kmid = kmid_ref[0]
    klo = klo_ref[0]
    kt = kt_ref[0]
    q = keys_ref[0, pl.ds(qi * Q, Q), :]
    q3 = q[:, 0:3]
    sqq = jnp.sum(q * q, axis=1, keepdims=True)
    sqk = jnp.sum(kt * kt, axis=0, keepdims=True)
    gram = jax.lax.dot_general(q, kt, (((1,), (0,)), ((), ())),
                               preferred_element_type=F32,
                               precision=jax.lax.Precision.DEFAULT)
    d2 = sqq + sqk - 2.0 * gram
    d2_scr[...] = jnp.maximum(d2, 0.0)
    iota = jax.lax.broadcasted_iota(jnp.int32, (Q, N), 1)

    def body(k, carry):
        d2v = d2_scr[...]
        m = jnp.min(d2v, axis=1, keepdims=True)
        sel = jnp.min(jnp.where(d2v == m, iota, N), axis=1,
                      keepdims=True)
        oh = iota == sel
        ohb = oh.astype(BF16)
        nb = (_dot(ohb, khi) + _dot(ohb, kmid)) + _dot(ohb, klo)
        row = jnp.concatenate(
            [q3, nb[:, 0:3], q3 - nb[:, 0:3], jnp.sqrt(m + 1e-12),
             jnp.zeros((Q, 6), F32)], axis=1)
        concat_ref[0, pl.ds(k, 1)] = row[None]
        d2_scr[...] = jnp.where(oh, BIG, d2v)
        return carry

    jax.lax.fori_loop(0, K, body, 0)
    flat = jnp.reshape(concat_ref[0], (K * Q, 16))
    m2 = _dott(flat, flat)
    sc = jnp.sum(flat, axis=0, keepdims=True)

    @pl.when(jnp.logical_and(b == 0, qi == 0))
    def _():
        mom_ref[...] = jnp.zeros_like(mom_ref)

    mom_ref[0:16, :] += m2
    mom_ref[16:17, :] += sc


def _p2_body(K, Q, concat_ref, feat_ref, w1_ref, b1_ref, w2_ref, b2_ref,
             sw1_ref, sw2_ref, m1w_ref, m1b_ref, o1g_ref, o1f_ref, o1b_ref,
             shw_ref, shb_ref,
             g2_ref, y1_ref, ys_ref, st1_ref, sts_ref):
    b = pl.program_id(0)
    qi = pl.program_id(1)
    flat = jnp.reshape(concat_ref[0], (K * Q, 16))
    enc1 = jnp.maximum(_dot(flat, w1_ref[...]) + b1_ref[...], 0.0)
    enc2 = jnp.maximum(_dot(flat, w2_ref[...]) + b2_ref[...], 0.0)

    def pool(enc, sw):
        a = _dot(enc, sw)
        ar = jnp.reshape(a, (K, Q, 64))
        er = jnp.reshape(enc, (K, Q, 64))
        mx = jnp.max(ar, axis=0, keepdims=True)
        ex = jnp.exp(ar - mx)
        s = ex / jnp.sum(ex, axis=0, keepdims=True)
        return jnp.sum(s * er, axis=0)

    g1 = pool(enc1, sw1_ref[...])
    g2 = pool(enc2, sw2_ref[...])
    g2_ref[0] = g2
    f = feat_ref[0]
    x0 = _dot(f, m1w_ref[...]) + m1b_ref[...]
    x0 = jnp.where(x0 >= 0, x0, 0.2 * x0)
    y1 = _dot(g1, o1g_ref[...]) + _dot(x0, o1f_ref[...]) + o1b_ref[...]
    y1_ref[0] = y1
    ys = _dot(f, shw_ref[...]) + shb_ref[...]
    ys_ref[0] = ys

    @pl.when(jnp.logical_and(b == 0, qi == 0))
    def _():
        st1_ref[...] = jnp.zeros_like(st1_ref)
        sts_ref[...] = jnp.zeros_like(sts_ref)

    st1_ref[0:1, :] += jnp.sum(y1, axis=0, keepdims=True)
    st1_ref[1:2, :] += jnp.sum(y1 * y1, axis=0, keepdims=True)
    sts_ref[0:1, :] += jnp.sum(ys, axis=0, keepdims=True)
    sts_ref[1:2, :] += jnp.sum(ys * ys, axis=0, keepdims=True)


def _p3_body(y1_ref, g2_ref, o2g_ref, o2f_ref, o2b_ref, sc1_ref, sh1_ref,
             y2_ref, st2_ref):
    b = pl.program_id(0)
    qi = pl.program_id(1)
    x1 = jnp.maximum(y1_ref[0] * sc1_ref[...] + sh1_ref[...], 0.0)
    y2 = _dot(g2_ref[0], o2g_ref[...]) + _dot(x1, o2f_ref[...]) + o2b_ref[...]
    y2_ref[0] = y2

    @pl.when(jnp.logical_and(b == 0, qi == 0))
    def _():
        st2_ref[...] = jnp.zeros_like(st2_ref)

    st2_ref[0:1, :] += jnp.sum(y2, axis=0, keepdims=True)
    st2_ref[1:2, :] += jnp.sum(y2 * y2, axis=0, keepdims=True)


def _p4_body(y2_ref, ys_ref, m2w_ref, m2b_ref, sc2_ref, sh2_ref, scs_ref,
             shs_ref, out_ref):
    x2 = jnp.maximum(y2_ref[0] * sc2_ref[...] + sh2_ref[...], 0.0)
    o = _dot(x2, m2w_ref[...]) + m2b_ref[...] + ys_ref[0] * scs_ref[...] \
        + shs_ref[...]
    out_ref[0] = jnp.where(o >= 0, o, 0.01 * o)


def kernel(coords, features, mlp1_w, mlp1_b, lse1_w, lse1_b, lse1_g, lse1_bt,
           pool1_sw, pool1_sb, pool1_ow, pool1_ob, pool1_g, pool1_bt,
           lse2_w, lse2_b, lse2_g, lse2_bt, pool2_sw, pool2_sb, pool2_ow,
           pool2_ob, pool2_g, pool2_bt, mlp2_w, mlp2_b, short_w, short_b,
           short_g, short_bt):
    B, N, _ = coords.shape
    K = 16
    h = mlp1_w.shape[0]
    d_in = mlp1_w.shape[1]
    d_out = pool1_sw.shape[0]
    d_fin = mlp2_w.shape[0]
    eps = 1e-6

    coords_pad = jnp.concatenate(
        [coords, jnp.zeros((B, N, 8 - coords.shape[2]), F32)], axis=2)
    coords_t = jnp.transpose(coords_pad, (0, 2, 1))
    feat_t = jnp.transpose(features[:, :, :, 0], (0, 2, 1))
    k_hi = coords_pad.astype(BF16)
    r1 = coords_pad - k_hi.astype(F32)
    k_mid = r1.astype(BF16)
    k_lo = (r1 - k_mid.astype(F32)).astype(BF16)

    Q1 = 512
    nb1 = N // Q1
    p1 = pl.pallas_call(
        functools.partial(_p1_body, K, Q1, N),
        grid=(B, nb1),
        in_specs=[
            pl.BlockSpec((1, N, 8), lambda b, q: (b, 0, 0)),
            pl.BlockSpec((1, 8, N), lambda b, q: (b, 0, 0)),
            pl.BlockSpec((1, N, 8), lambda b, q: (b, 0, 0)),
            pl.BlockSpec((1, N, 8), lambda b, q: (b, 0, 0)),
            pl.BlockSpec((1, N, 8), lambda b, q: (b, 0, 0)),
        ],
        out_specs=[
            pl.BlockSpec((1, K, Q1, 16), lambda b, q: (b, 0, q, 0)),
            pl.BlockSpec((24, 16), lambda b, q: (0, 0)),
        ],
        out_shape=[
            jax.ShapeDtypeStruct((B, K, N, 16), F32),
            jax.ShapeDtypeStruct((24, 16), F32),
        ],
        scratch_shapes=[
            pltpu.VMEM((Q1, N), F32),
        ],
    )
    concat, mom = p1(coords_pad, coords_t, k_hi, k_mid, k_lo)

    M = B * N * K
    mu = mom[16, 0:10] / M
    m2 = mom[0:10, 0:10] / M
    cc = m2 - jnp.outer(mu, mu)

    def fold(w, bb, g, bt):
        mean = w @ mu + bb
        var = jnp.sum((w @ cc) * w, axis=1)
        sc = g / jnp.sqrt(var + eps)
        weff = jnp.zeros((16, h), F32).at[0:10, :].set((w * sc[:, None]).T)
        beff = (bb - mean) * sc + bt
        return weff, beff[None, :]

    w1eff, b1eff = fold(lse1_w, lse1_b, lse1_g, lse1_bt)
    w2eff, b2eff = fold(lse2_w, lse2_b, lse2_g, lse2_bt)

    Q2 = 512
    nb2 = N // Q2
    cst = lambda b, q: (0, 0)
    p2 = pl.pallas_call(
        functools.partial(_p2_body, K, Q2),
        grid=(B, nb2),
        in_specs=[
            pl.BlockSpec((1, K, Q2, 16), lambda b, q: (b, 0, q, 0)),
            pl.BlockSpec((1, Q2, d_in), lambda b, q: (b, q, 0)),
            pl.BlockSpec((16, h), cst), pl.BlockSpec((1, h), cst),
            pl.BlockSpec((16, h), cst), pl.BlockSpec((1, h), cst),
            pl.BlockSpec((h, h), cst), pl.BlockSpec((h, h), cst),
            pl.BlockSpec((d_in, h), cst), pl.BlockSpec((1, h), cst),
            pl.BlockSpec((h, h), cst), pl.BlockSpec((h, h), cst),
            pl.BlockSpec((1, h), cst),
            pl.BlockSpec((d_in, d_fin), cst), pl.BlockSpec((1, d_fin), cst),
        ],
        out_specs=[
            pl.BlockSpec((1, Q2, h), lambda b, q: (b, q, 0)),
            pl.BlockSpec((1, Q2, h), lambda b, q: (b, q, 0)),
            pl.BlockSpec((1, Q2, d_fin), lambda b, q: (b, q, 0)),
            pl.BlockSpec((8, h), cst),
            pl.BlockSpec((8, d_fin), cst),
        ],
        out_shape=[
            jax.ShapeDtypeStruct((B, N, h), F32),
            jax.ShapeDtypeStruct((B, N, h), F32),
            jax.ShapeDtypeStruct((B, N, d_fin), F32),
            jax.ShapeDtypeStruct((8, h), F32),
            jax.ShapeDtypeStruct((8, d_fin), F32),
        ],
    )
    g2, y1, ys, st1, sts = p2(
        concat, feat_t,
        w1eff, b1eff, w2eff, b2eff,
        pool1_sw[0:h, 0:h].T, pool2_sw[0:h, 0:h].T,
        mlp1_w.T, mlp1_b[None, :],
        pool1_ow[:, 0:h].T, pool1_ow[:, h:].T, pool1_ob[None, :],
        short_w.T, short_b[None, :])

    def bnstats(st, g, bt, cnt):
        mean = st[0] / cnt
        var = st[1] / cnt - mean * mean
        sc = g / jnp.sqrt(var + eps)
        return sc[None, :], (bt - mean * sc)[None, :]

    sc1, sh1 = bnstats(st1, pool1_g, pool1_bt, B * N)
    scs, shs = bnstats(sts, short_g, short_bt, B * N)

    Q3 = 2048
    nb3 = N // Q3
    p3 = pl.pallas_call(
        _p3_body,
        grid=(B, nb3),
        in_specs=[
            pl.BlockSpec((1, Q3, h), lambda b, q: (b, q, 0)),
            pl.BlockSpec((1, Q3, h), lambda b, q: (b, q, 0)),
            pl.BlockSpec((h, d_out), cst), pl.BlockSpec((h, d_out), cst),
            pl.BlockSpec((1, d_out), cst),
            pl.BlockSpec((1, h), cst), pl.BlockSpec((1, h), cst),
        ],
        out_specs=[
            pl.BlockSpec((1, Q3, d_out), lambda b, q: (b, q, 0)),
            pl.BlockSpec((8, d_out), cst),
        ],
        out_shape=[
            jax.ShapeDtypeStruct((B, N, d_out), F32),
            jax.ShapeDtypeStruct((8, d_out), F32),
        ],
    )
    y2, st2 = p3(y1, g2, pool2_ow[:, 0:h].T, pool2_ow[:, h:].T,
                 pool2_ob[None, :], sc1, sh1)

    sc2, sh2 = bnstats(st2, pool2_g, pool2_bt, B * N)

    p4 = pl.pallas_call(
        _p4_body,
        grid=(B, nb3),
        in_specs=[
            pl.BlockSpec((1, Q3, d_out), lambda b, q: (b, q, 0)),
            pl.BlockSpec((1, Q3, d_fin), lambda b, q: (b, q, 0)),
            pl.BlockSpec((d_out, d_fin), cst), pl.BlockSpec((1, d_fin), cst),
            pl.BlockSpec((1, d_out), cst), pl.BlockSpec((1, d_out), cst),
            pl.BlockSpec((1, d_fin), cst), pl.BlockSpec((1, d_fin), cst),
        ],
        out_specs=[pl.BlockSpec((1, Q3, d_fin), lambda b, q: (b, q, 0))],
        out_shape=[jax.ShapeDtypeStruct((B, N, d_fin), F32)],
    )
    (out,) = p4(y2, ys, mlp2_w.T, mlp2_b[None, :], sc2, sh2, scs, shs)

    return jnp.transpose(out, (0, 2, 1))[:, :, :, None]


# speedup vs baseline: 8.0655x; 1.0800x over previous
import functools

import jax
import jax.numpy as jnp
from jax.experimental import pallas as pl
from jax.experimental.pallas import tpu as pltpu

F32 = jnp.float32
BF16 = jnp.bfloat16
BIG = 3.0e38


def _dot(a, b):
    return jax.lax.dot_general(a, b, (((1,), (0,)), ((), ())),
                               preferred_element_type=F32,
                               precision=jax.lax.Precision.DEFAULT)


def _dott(a, b):
    return jax.lax.dot_general(a, b, (((0,), (0,)), ((), ())),
                               preferred_element_type=F32,
                               precision=jax.lax.Precision.HIGHEST)


def _p1_body(K, Q, N, keys_ref, kt_ref, k24_ref, concat_ref, mom_ref,
             d2_scr):
    b = pl.program_id(0)
    qi = pl.program_id(1)
    k24 = k24_ref[0]
    kt = kt_ref[0]
    q = keys_ref[0, pl.ds(qi * Q, Q), :]
    q3 = q[:, 0:3]
    sqq = jnp.sum(q * q, axis=1, keepdims=True)
    sqk = jnp.sum(kt * kt, axis=0, keepdims=True)
    gram = jax.lax.dot_general(q, kt, (((1,), (0,)), ((), ())),
                               preferred_element_type=F32,
                               precision=jax.lax.Precision.DEFAULT)
    d2_scr[...] = jnp.maximum(sqq + sqk - 2.0 * gram, 0.0)
    iota = jax.lax.broadcasted_iota(jnp.int32, (Q, N), 1)

    def body(k, carry):
        m_prev, sel_prev = carry
        d2v = d2_scr[...]
        cand = jnp.logical_or(
            d2v > m_prev,
            jnp.logical_and(d2v == m_prev, iota > sel_prev))
        m = jnp.min(jnp.where(cand, d2v, BIG), axis=1, keepdims=True)
        sel = jnp.min(jnp.where(jnp.logical_and(cand, d2v == m), iota, N),
                      axis=1, keepdims=True)
        ohb = (iota == sel).astype(BF16)
        nb24 = _dot(ohb, k24)
        nb = (nb24[:, 0:3] + nb24[:, 8:11]) + nb24[:, 16:19]
        row = jnp.concatenate(
            [q3, nb, q3 - nb, jnp.sqrt(m + 1e-12),
             jnp.zeros((Q, 6), F32)], axis=1)
        concat_ref[0, pl.ds(k, 1)] = row[None]
        return (m, sel)

    jax.lax.fori_loop(0, K, body,
                      (jnp.full((Q, 1), -1.0, F32),
                       jnp.full((Q, 1), N, jnp.int32)))
    flat = jnp.reshape(concat_ref[0], (K * Q, 16))
    m2 = _dott(flat, flat)
    sc = jnp.sum(flat, axis=0, keepdims=True)

    @pl.when(jnp.logical_and(b == 0, qi == 0))
    def _():
        mom_ref[...] = jnp.zeros_like(mom_ref)

    mom_ref[0:16, :] += m2
    mom_ref[16:17, :] += sc


def _p2_body(K, Q, concat_ref, feat_ref, w1_ref, b1_ref, w2_ref, b2_ref,
             sw1_ref, sw2_ref, m1w_ref, m1b_ref, o1g_ref, o1f_ref, o1b_ref,
             shw_ref, shb_ref,
             g2_ref, y1_ref, ys_ref, st1_ref, sts_ref):
    b = pl.program_id(0)
    qi = pl.program_id(1)
    flat = jnp.reshape(concat_ref[0], (K * Q, 16))
    enc1 = jnp.maximum(_dot(flat, w1_ref[...]) + b1_ref[...], 0.0)
    enc2 = jnp.maximum(_dot(flat, w2_ref[...]) + b2_ref[...], 0.0)

    def pool(enc, sw):
        a = _dot(enc, sw)
        ar = jnp.reshape(a, (K, Q, 64))
        er = jnp.reshape(enc, (K, Q, 64))
        mx = jnp.max(ar, axis=0, keepdims=True)
        ex = jnp.exp(ar - mx)
        s = ex / jnp.sum(ex, axis=0, keepdims=True)
        return jnp.sum(s * er, axis=0)

    g1 = pool(enc1, sw1_ref[...])
    g2 = pool(enc2, sw2_ref[...])
    g2_ref[0] = g2
    f = feat_ref[0]
    x0 = _dot(f, m1w_ref[...]) + m1b_ref[...]
    x0 = jnp.where(x0 >= 0, x0, 0.2 * x0)
    y1 = _dot(g1, o1g_ref[...]) + _dot(x0, o1f_ref[...]) + o1b_ref[...]
    y1_ref[0] = y1
    ys = _dot(f, shw_ref[...]) + shb_ref[...]
    ys_ref[0] = ys

    @pl.when(jnp.logical_and(b == 0, qi == 0))
    def _():
        st1_ref[...] = jnp.zeros_like(st1_ref)
        sts_ref[...] = jnp.zeros_like(sts_ref)

    st1_ref[0:1, :] += jnp.sum(y1, axis=0, keepdims=True)
    st1_ref[1:2, :] += jnp.sum(y1 * y1, axis=0, keepdims=True)
    sts_ref[0:1, :] += jnp.sum(ys, axis=0, keepdims=True)
    sts_ref[1:2, :] += jnp.sum(ys * ys, axis=0, keepdims=True)


def _p3_body(y1_ref, g2_ref, o2g_ref, o2f_ref, o2b_ref, sc1_ref, sh1_ref,
             y2_ref, st2_ref):
    b = pl.program_id(0)
    qi = pl.program_id(1)
    x1 = jnp.maximum(y1_ref[0] * sc1_ref[...] + sh1_ref[...], 0.0)
    y2 = _dot(g2_ref[0], o2g_ref[...]) + _dot(x1, o2f_ref[...]) + o2b_ref[...]
    y2_ref[0] = y2

    @pl.when(jnp.logical_and(b == 0, qi == 0))
    def _():
        st2_ref[...] = jnp.zeros_like(st2_ref)

    st2_ref[0:1, :] += jnp.sum(y2, axis=0, keepdims=True)
    st2_ref[1:2, :] += jnp.sum(y2 * y2, axis=0, keepdims=True)


def _p4_body(y2_ref, ys_ref, m2w_ref, m2b_ref, sc2_ref, sh2_ref, scs_ref,
             shs_ref, out_ref):
    x2 = jnp.maximum(y2_ref[0] * sc2_ref[...] + sh2_ref[...], 0.0)
    o = _dot(x2, m2w_ref[...]) + m2b_ref[...] + ys_ref[0] * scs_ref[...] \
        + shs_ref[...]
    out_ref[0] = jnp.where(o >= 0, o, 0.01 * o)


def kernel(coords, features, mlp1_w, mlp1_b, lse1_w, lse1_b, lse1_g, lse1_bt,
           pool1_sw, pool1_sb, pool1_ow, pool1_ob, pool1_g, pool1_bt,
           lse2_w, lse2_b, lse2_g, lse2_bt, pool2_sw, pool2_sb, pool2_ow,
           pool2_ob, pool2_g, pool2_bt, mlp2_w, mlp2_b, short_w, short_b,
           short_g, short_bt):
    B, N, _ = coords.shape
    K = 16
    h = mlp1_w.shape[0]
    d_in = mlp1_w.shape[1]
    d_out = pool1_sw.shape[0]
    d_fin = mlp2_w.shape[0]
    eps = 1e-6

    coords_pad = jnp.concatenate(
        [coords, jnp.zeros((B, N, 8 - coords.shape[2]), F32)], axis=2)
    coords_t = jnp.transpose(coords_pad, (0, 2, 1))
    feat_t = jnp.transpose(features[:, :, :, 0], (0, 2, 1))
    k_hi = coords_pad.astype(BF16)
    r1 = coords_pad - k_hi.astype(F32)
    k_mid = r1.astype(BF16)
    k_lo = (r1 - k_mid.astype(F32)).astype(BF16)
    k24 = jnp.concatenate([k_hi, k_mid, k_lo], axis=2)

    Q1 = 512
    nb1 = N // Q1
    p1 = pl.pallas_call(
        functools.partial(_p1_body, K, Q1, N),
        grid=(B, nb1),
        in_specs=[
            pl.BlockSpec((1, N, 8), lambda b, q: (b, 0, 0)),
            pl.BlockSpec((1, 8, N), lambda b, q: (b, 0, 0)),
            pl.BlockSpec((1, N, 24), lambda b, q: (b, 0, 0)),
        ],
        out_specs=[
            pl.BlockSpec((1, K, Q1, 16), lambda b, q: (b, 0, q, 0)),
            pl.BlockSpec((24, 16), lambda b, q: (0, 0)),
        ],
        out_shape=[
            jax.ShapeDtypeStruct((B, K, N, 16), F32),
            jax.ShapeDtypeStruct((24, 16), F32),
        ],
        scratch_shapes=[
            pltpu.VMEM((Q1, N), F32),
        ],
    )
    concat, mom = p1(coords_pad, coords_t, k24)

    M = B * N * K
    mu = mom[16, 0:10] / M
    m2 = mom[0:10, 0:10] / M
    cc = m2 - jnp.outer(mu, mu)

    def fold(w, bb, g, bt):
        mean = w @ mu + bb
        var = jnp.sum((w @ cc) * w, axis=1)
        sc = g / jnp.sqrt(var + eps)
        weff = jnp.zeros((16, h), F32).at[0:10, :].set((w * sc[:, None]).T)
        beff = (bb - mean) * sc + bt
        return weff, beff[None, :]

    w1eff, b1eff = fold(lse1_w, lse1_b, lse1_g, lse1_bt)
    w2eff, b2eff = fold(lse2_w, lse2_b, lse2_g, lse2_bt)

    Q2 = 512
    nb2 = N // Q2
    cst = lambda b, q: (0, 0)
    p2 = pl.pallas_call(
        functools.partial(_p2_body, K, Q2),
        grid=(B, nb2),
        in_specs=[
            pl.BlockSpec((1, K, Q2, 16), lambda b, q: (b, 0, q, 0)),
            pl.BlockSpec((1, Q2, d_in), lambda b, q: (b, q, 0)),
            pl.BlockSpec((16, h), cst), pl.BlockSpec((1, h), cst),
            pl.BlockSpec((16, h), cst), pl.BlockSpec((1, h), cst),
            pl.BlockSpec((h, h), cst), pl.BlockSpec((h, h), cst),
            pl.BlockSpec((d_in, h), cst), pl.BlockSpec((1, h), cst),
            pl.BlockSpec((h, h), cst), pl.BlockSpec((h, h), cst),
            pl.BlockSpec((1, h), cst),
            pl.BlockSpec((d_in, d_fin), cst), pl.BlockSpec((1, d_fin), cst),
        ],
        out_specs=[
            pl.BlockSpec((1, Q2, h), lambda b, q: (b, q, 0)),
            pl.BlockSpec((1, Q2, h), lambda b, q: (b, q, 0)),
            pl.BlockSpec((1, Q2, d_fin), lambda b, q: (b, q, 0)),
            pl.BlockSpec((8, h), cst),
            pl.BlockSpec((8, d_fin), cst),
        ],
        out_shape=[
            jax.ShapeDtypeStruct((B, N, h), F32),
            jax.ShapeDtypeStruct((B, N, h), F32),
            jax.ShapeDtypeStruct((B, N, d_fin), F32),
            jax.ShapeDtypeStruct((8, h), F32),
            jax.ShapeDtypeStruct((8, d_fin), F32),
        ],
    )
    g2, y1, ys, st1, sts = p2(
        concat, feat_t,
        w1eff, b1eff, w2eff, b2eff,
        pool1_sw[0:h, 0:h].T, pool2_sw[0:h, 0:h].T,
        mlp1_w.T, mlp1_b[None, :],
        pool1_ow[:, 0:h].T, pool1_ow[:, h:].T, pool1_ob[None, :],
        short_w.T, short_b[None, :])

    def bnstats(st, g, bt, cnt):
        mean = st[0] / cnt
        var = st[1] / cnt - mean * mean
        sc = g / jnp.sqrt(var + eps)
        return sc[None, :], (bt - mean * sc)[None, :]

    sc1, sh1 = bnstats(st1, pool1_g, pool1_bt, B * N)
    scs, shs = bnstats(sts, short_g, short_bt, B * N)

    Q3 = 2048
    nb3 = N // Q3
    p3 = pl.pallas_call(
        _p3_body,
        grid=(B, nb3),
        in_specs=[
            pl.BlockSpec((1, Q3, h), lambda b, q: (b, q, 0)),
            pl.BlockSpec((1, Q3, h), lambda b, q: (b, q, 0)),
            pl.BlockSpec((h, d_out), cst), pl.BlockSpec((h, d_out), cst),
            pl.BlockSpec((1, d_out), cst),
            pl.BlockSpec((1, h), cst), pl.BlockSpec((1, h), cst),
        ],
        out_specs=[
            pl.BlockSpec((1, Q3, d_out), lambda b, q: (b, q, 0)),
            pl.BlockSpec((8, d_out), cst),
        ],
        out_shape=[
            jax.ShapeDtypeStruct((B, N, d_out), F32),
            jax.ShapeDtypeStruct((8, d_out), F32),
        ],
    )
    y2, st2 = p3(y1, g2, pool2_ow[:, 0:h].T, pool2_ow[:, h:].T,
                 pool2_ob[None, :], sc1, sh1)

    sc2, sh2 = bnstats(st2, pool2_g, pool2_bt, B * N)

    p4 = pl.pallas_call(
        _p4_body,
        grid=(B, nb3),
        in_specs=[
            pl.BlockSpec((1, Q3, d_out), lambda b, q: (b, q, 0)),
            pl.BlockSpec((1, Q3, d_fin), lambda b, q: (b, q, 0)),
            pl.BlockSpec((d_out, d_fin), cst), pl.BlockSpec((1, d_fin), cst),
            pl.BlockSpec((1, d_out), cst), pl.BlockSpec((1, d_out), cst),
            pl.BlockSpec((1, d_fin), cst), pl.BlockSpec((1, d_fin), cst),
        ],
        out_specs=[pl.BlockSpec((1, Q3, d_fin), lambda b, q: (b, q, 0))],
        out_shape=[jax.ShapeDtypeStruct((B, N, d_fin), F32)],
    )
    (out,) = p4(y2, ys, mlp2_w.T, mlp2_b[None, :], sc2, sh2, scs, shs)

    return jnp.transpose(out, (0, 2, 1))[:, :, :, None]
